# baseline (speedup 1.0000x reference)
import functools

import jax
import jax.numpy as jnp
from jax import lax
from jax.experimental import pallas as pl
from jax.experimental.pallas import tpu as pltpu
from jax.experimental.pallas import tpu_sc as plsc

N_FEAT = 26
VOCAB = 100001
VOCAB_P = 100032
EMB = 16
B = 1024
L = 200
H = 128
D = N_FEAT * EMB
TOP = 32
D_P = 512

NC = 2
NS = 16
NW = NC * NS
FH = N_FEAT // 2
D_H = FH * EMB
D_HP = 256
SLOT = D_HP // EMB
N_TOT = L * B * FH
N_OUT = L * B * SLOT
ROWS_PER_W = N_TOT // NW
KK = 10
CSZ = KK * 128
N_CHUNK = ROWS_PER_W // CSZ
CHUNKS_PER_F = (B * L) // CSZ


def _sc_gather_body(table_hbm, idx_hbm, dst_hbm, out_hbm,
                    idx_v, dst_v, rows_v, sem, sem_s):
    wid = lax.axis_index("s") * NC + lax.axis_index("c")
    row0 = wid * (ROWS_PER_W // 128)

    H1 = KK // 2

    def chunk(cj, carry):
        f = (wid * N_CHUNK + cj) // CHUNKS_PER_F
        pltpu.sync_copy(idx_hbm.at[pl.ds(row0 + cj * KK, KK)], idx_v)
        pltpu.sync_copy(dst_hbm.at[pl.ds(row0 + cj * KK, KK)], dst_v)

        def gather(jj):
            return pltpu.async_copy(
                table_hbm.at[f].at[idx_v.at[jj]],
                rows_v.at[pl.ds(jj * 128, 128)], sem)

        def scatter(jj):
            return pltpu.async_copy(
                rows_v.at[pl.ds(jj * 128, 128)],
                out_hbm.at[dst_v.at[jj]], sem_s)

        g = [gather(jj) for jj in range(KK)]
        for h in g:
            h.wait()
        s = [scatter(jj) for jj in range(KK)]
        for h in s:
            h.wait()
        return carry

    lax.fori_loop(0, N_CHUNK, chunk, 0)


def _sc_gather(flat_table, idx_w, dst_w):
    mesh = plsc.VectorSubcoreMesh(core_axis_name="c", subcore_axis_name="s")
    k = functools.partial(
        pl.kernel,
        mesh=mesh,
        out_type=jax.ShapeDtypeStruct((N_OUT, EMB), jnp.float32),
        scratch_types=[
            pltpu.VMEM((KK, 128), jnp.int32),
            pltpu.VMEM((KK, 128), jnp.int32),
            pltpu.VMEM((CSZ, EMB), jnp.float32),
            pltpu.SemaphoreType.DMA,
            pltpu.SemaphoreType.DMA,
        ],
        compiler_params=pltpu.CompilerParams(use_tc_tiling_on_sc=False),
    )(_sc_gather_body)
    return k(flat_table, idx_w, dst_w)


def _rnn_body(xfa_ref, xba_ref, xfb_ref, xbb_ref,
              wihfa, wihfb, whhf, bihf, bhhf,
              wihba, wihbb, whhb, bihb, bhhb, w1t, b1, w2t, b2,
              out_ref, hf, hb, mxf, mxb, smf, smb):
    t = pl.program_id(0)

    @pl.when(t == 0)
    def _init():
        z = jnp.zeros((B, H), jnp.float32)
        ninf = jnp.full((B, H), -jnp.inf, jnp.float32)
        hf[...] = z
        hb[...] = z
        smf[...] = z
        smb[...] = z
        mxf[...] = ninf
        mxb[...] = ninf

    def gru_step(xa, xb, h, wiha, wihb_, whh, bih, bhh):
        gi = (jnp.dot(xa.astype(jnp.bfloat16), wiha[...],
                      preferred_element_type=jnp.float32)
              + jnp.dot(xb.astype(jnp.bfloat16), wihb_[...],
                        preferred_element_type=jnp.float32) + bih[...])
        gh = jnp.dot(h.astype(jnp.bfloat16), whh[...],
                     preferred_element_type=jnp.float32) + bhh[...]
        i_r, i_z, i_n = gi[:, :H], gi[:, H:2 * H], gi[:, 2 * H:]
        h_r, h_z, h_n = gh[:, :H], gh[:, H:2 * H], gh[:, 2 * H:]
        r = jax.nn.sigmoid(i_r + h_r)
        z = jax.nn.sigmoid(i_z + h_z)
        n = jnp.tanh(i_n + r * h_n)
        return (1.0 - z) * n + z * h

    lane = lax.broadcasted_iota(jnp.int32, (B, D_HP), 1)
    msk = lambda v: jnp.where(lane < D_H, v, 0.0)
    hf_new = gru_step(msk(xfa_ref[0]), msk(xfb_ref[0]), hf[...],
                      wihfa, wihfb, whhf, bihf, bhhf)
    hb_new = gru_step(msk(xba_ref[0]), msk(xbb_ref[0]), hb[...],
                      wihba, wihbb, whhb, bihb, bhhb)
    hf[...] = hf_new
    hb[...] = hb_new
    mxf[...] = jnp.maximum(mxf[...], hf_new)
    mxb[...] = jnp.maximum(mxb[...], hb_new)
    smf[...] = smf[...] + hf_new
    smb[...] = smb[...] + hb_new

    @pl.when(t == L - 1)
    def _final():
        inv_l = jnp.float32(1.0 / L)
        combined = jnp.concatenate(
            [mxf[...], mxb[...], smf[...] * inv_l, smb[...] * inv_l], axis=1)
        h1 = jnp.maximum(
            jnp.dot(combined, w1t[...], preferred_element_type=jnp.float32)
            + b1[...], 0.0)
        logit = jnp.dot(h1, w2t[...], preferred_element_type=jnp.float32) + b2[...]
        out_ref[...] = logit


def _rnn_call(xa, xb, wihfa, wihfb, whhf_t, bihf, bhhf,
              wihba, wihbb, whhb_t, bihb, bhhb,
              w1t, b1, w2t, b2, interpret=False):
    full = lambda s: pl.BlockSpec(s, lambda t: (0,) * len(s))
    wih = full((D_HP, 3 * H))
    return pl.pallas_call(
        _rnn_body,
        grid=(L,),
        in_specs=[
            pl.BlockSpec((1, B, D_HP), lambda t: (t, 0, 0)),
            pl.BlockSpec((1, B, D_HP), lambda t: (L - 1 - t, 0, 0)),
            pl.BlockSpec((1, B, D_HP), lambda t: (t, 0, 0)),
            pl.BlockSpec((1, B, D_HP), lambda t: (L - 1 - t, 0, 0)),
            wih, wih, full((H, 3 * H)), full((1, 3 * H)), full((1, 3 * H)),
            wih, wih, full((H, 3 * H)), full((1, 3 * H)), full((1, 3 * H)),
            full((4 * H, TOP)), full((1, TOP)), full((TOP, 1)), full((1, 1)),
        ],
        out_specs=pl.BlockSpec((B, 1), lambda t: (0, 0)),
        out_shape=jax.ShapeDtypeStruct((B, 1), jnp.float32),
        scratch_shapes=[pltpu.VMEM((B, H), jnp.float32)] * 6,
        compiler_params=pltpu.CompilerParams(
            dimension_semantics=("arbitrary",)),
        interpret=interpret,
    )(xa, xa, xb, xb, wihfa, wihfb, whhf_t, bihf, bhhf,
      wihba, wihbb, whhb_t, bihb, bhhb, w1t, b1, w2t, b2)


def kernel(transactions_cat_features, emb_tables, W_ih_f, W_hh_f, b_ih_f,
           b_hh_f, W_ih_b, W_hh_b, b_ih_b, b_hh_b, W1, b1, W2, b2):
    f_ax = jnp.arange(FH, dtype=jnp.int32)[:, None, None]
    b_ax = jnp.arange(B, dtype=jnp.int32)[None, :, None]
    t_ax = jnp.arange(L, dtype=jnp.int32)[None, None, :]
    dst_nat = t_ax * (B * SLOT) + b_ax * SLOT + f_ax
    dst_w = jnp.broadcast_to(
        dst_nat, (FH, B, L)).reshape(N_TOT // 128, 128)

    xs = []
    for h in range(2):
        tab = jnp.pad(emb_tables[h * FH:(h + 1) * FH],
                      ((0, 0), (0, VOCAB_P - VOCAB), (0, 0)))
        idx_w = transactions_cat_features[h * FH:(h + 1) * FH].reshape(
            N_TOT // 128, 128)
        xs.append(_sc_gather(tab, idx_w, dst_w).reshape(L, B, D_HP))

    bf = jnp.bfloat16
    wa = lambda w: jnp.pad(w.T[:D_H], ((0, D_HP - D_H), (0, 0))).astype(bf)
    wb = lambda w: jnp.pad(w.T[D_H:], ((0, D_HP - D_H), (0, 0))).astype(bf)
    logit = _rnn_call(
        xs[0], xs[1],
        wa(W_ih_f), wb(W_ih_f), W_hh_f.T.astype(bf), b_ih_f.reshape(1, -1),
        b_hh_f.reshape(1, -1),
        wa(W_ih_b), wb(W_ih_b), W_hh_b.T.astype(bf), b_ih_b.reshape(1, -1),
        b_hh_b.reshape(1, -1),
        W1.T, b1.reshape(1, -1), W2.T, b2.reshape(1, -1))
    return logit


# speedup vs baseline: 1.0688x; 1.0688x over previous
import functools

import jax
import jax.numpy as jnp
from jax import lax
from jax.experimental import pallas as pl
from jax.experimental.pallas import tpu as pltpu
from jax.experimental.pallas import tpu_sc as plsc

N_FEAT = 26
VOCAB = 100001
VOCAB_P = 100032
EMB = 16
B = 1024
L = 200
H = 128
D = N_FEAT * EMB
TOP = 32
D_P = 512

NC = 2
NS = 16
NW = NC * NS
N_TOT = L * B * N_FEAT
SLOT = 32
N_OUT = L * B * SLOT
ROWS_PER_W = N_TOT // NW
KK = 10
CSZ = KK * 128
N_CHUNK = ROWS_PER_W // CSZ
CHUNKS_PER_F = (B * L) // CSZ


def _sc_gather_body(table_hbm, idx_hbm, dst_hbm, out_hbm,
                    idx_v, dst_v, rows_v, sem, sem_s):
    wid = lax.axis_index("s") * NC + lax.axis_index("c")
    row0 = wid * (ROWS_PER_W // 128)

    H1 = KK // 2

    def chunk(cj, carry):
        f = (wid * N_CHUNK + cj) // CHUNKS_PER_F
        pltpu.sync_copy(idx_hbm.at[pl.ds(row0 + cj * KK, KK)], idx_v)
        pltpu.sync_copy(dst_hbm.at[pl.ds(row0 + cj * KK, KK)], dst_v)

        def gather(jj):
            return pltpu.async_copy(
                table_hbm.at[f].at[idx_v.at[jj]],
                rows_v.at[pl.ds(jj * 128, 128)], sem)

        def scatter(jj):
            return pltpu.async_copy(
                rows_v.at[pl.ds(jj * 128, 128)],
                out_hbm.at[dst_v.at[jj]], sem_s)

        g = [gather(jj) for jj in range(KK)]
        for h in g:
            h.wait()
        s = [scatter(jj) for jj in range(KK)]
        for h in s:
            h.wait()
        return carry

    lax.fori_loop(0, N_CHUNK, chunk, 0)


def _sc_gather(flat_table, idx_w, dst_w):
    mesh = plsc.VectorSubcoreMesh(core_axis_name="c", subcore_axis_name="s")
    k = functools.partial(
        pl.kernel,
        mesh=mesh,
        out_type=jax.ShapeDtypeStruct((N_OUT, EMB), jnp.float32),
        scratch_types=[
            pltpu.VMEM((KK, 128), jnp.int32),
            pltpu.VMEM((KK, 128), jnp.int32),
            pltpu.VMEM((CSZ, EMB), jnp.float32),
            pltpu.SemaphoreType.DMA,
            pltpu.SemaphoreType.DMA,
        ],
        compiler_params=pltpu.CompilerParams(use_tc_tiling_on_sc=False),
    )(_sc_gather_body)
    return k(flat_table, idx_w, dst_w)


def _rnn_body(xf_ref, xb_ref, wihf, whhf, bihf, bhhf,
              wihb, whhb, bihb, bhhb, w1t, b1, w2t, b2,
              out_ref, hf, hb, mxf, mxb, smf, smb):
    t = pl.program_id(0)

    @pl.when(t == 0)
    def _init():
        z = jnp.zeros((B, H), jnp.float32)
        ninf = jnp.full((B, H), -jnp.inf, jnp.float32)
        hf[...] = z
        hb[...] = z
        smf[...] = z
        smb[...] = z
        mxf[...] = ninf
        mxb[...] = ninf

    def gru_step(x, h, wih, whh, bih, bhh):
        gi = jnp.dot(x.astype(jnp.bfloat16), wih[...],
                     preferred_element_type=jnp.float32) + bih[...]
        gh = jnp.dot(h.astype(jnp.bfloat16), whh[...],
                     preferred_element_type=jnp.float32) + bhh[...]
        i_r, i_z, i_n = gi[:, :H], gi[:, H:2 * H], gi[:, 2 * H:]
        h_r, h_z, h_n = gh[:, :H], gh[:, H:2 * H], gh[:, 2 * H:]
        r = jax.nn.sigmoid(i_r + h_r)
        z = jax.nn.sigmoid(i_z + h_z)
        n = jnp.tanh(i_n + r * h_n)
        return (1.0 - z) * n + z * h

    lane = lax.broadcasted_iota(jnp.int32, (B, D_P), 1)
    xf = jnp.where(lane < D, xf_ref[0], 0.0)
    xb = jnp.where(lane < D, xb_ref[0], 0.0)
    hf_new = gru_step(xf, hf[...], wihf, whhf, bihf, bhhf)
    hb_new = gru_step(xb, hb[...], wihb, whhb, bihb, bhhb)
    hf[...] = hf_new
    hb[...] = hb_new
    mxf[...] = jnp.maximum(mxf[...], hf_new)
    mxb[...] = jnp.maximum(mxb[...], hb_new)
    smf[...] = smf[...] + hf_new
    smb[...] = smb[...] + hb_new

    @pl.when(t == L - 1)
    def _final():
        inv_l = jnp.float32(1.0 / L)
        combined = jnp.concatenate(
            [mxf[...], mxb[...], smf[...] * inv_l, smb[...] * inv_l], axis=1)
        h1 = jnp.maximum(
            jnp.dot(combined, w1t[...], preferred_element_type=jnp.float32)
            + b1[...], 0.0)
        logit = jnp.dot(h1, w2t[...], preferred_element_type=jnp.float32) + b2[...]
        out_ref[...] = logit


def _rnn_call(x, wihf_t, whhf_t, bihf, bhhf, wihb_t, whhb_t, bihb, bhhb,
              w1t, b1, w2t, b2, interpret=False):
    full = lambda s: pl.BlockSpec(s, lambda t: (0,) * len(s))
    return pl.pallas_call(
        _rnn_body,
        grid=(L,),
        in_specs=[
            pl.BlockSpec((1, B, D_P), lambda t: (t, 0, 0)),
            pl.BlockSpec((1, B, D_P), lambda t: (L - 1 - t, 0, 0)),
            full((D_P, 3 * H)), full((H, 3 * H)), full((1, 3 * H)), full((1, 3 * H)),
            full((D_P, 3 * H)), full((H, 3 * H)), full((1, 3 * H)), full((1, 3 * H)),
            full((4 * H, TOP)), full((1, TOP)), full((TOP, 1)), full((1, 1)),
        ],
        out_specs=pl.BlockSpec((B, 1), lambda t: (0, 0)),
        out_shape=jax.ShapeDtypeStruct((B, 1), jnp.float32),
        scratch_shapes=[pltpu.VMEM((B, H), jnp.float32)] * 6,
        compiler_params=pltpu.CompilerParams(
            dimension_semantics=("arbitrary",)),
        interpret=interpret,
    )(x, x, wihf_t, whhf_t, bihf, bhhf, wihb_t, whhb_t, bihb, bhhb,
      w1t, b1, w2t, b2)


def kernel(transactions_cat_features, emb_tables, W_ih_f, W_hh_f, b_ih_f,
           b_hh_f, W_ih_b, W_hh_b, b_ih_b, b_hh_b, W1, b1, W2, b2):
    table_p = jnp.pad(emb_tables, ((0, 0), (0, VOCAB_P - VOCAB), (0, 0)))
    f_ax = jnp.arange(N_FEAT, dtype=jnp.int32)[:, None, None]
    b_ax = jnp.arange(B, dtype=jnp.int32)[None, :, None]
    t_ax = jnp.arange(L, dtype=jnp.int32)[None, None, :]
    dst_nat = t_ax * (B * SLOT) + b_ax * SLOT + f_ax
    idx_w = transactions_cat_features.reshape(N_TOT // 128, 128)
    dst_w = jnp.broadcast_to(
        dst_nat, transactions_cat_features.shape).reshape(N_TOT // 128, 128)

    x = _sc_gather(table_p, idx_w, dst_w).reshape(L, B, D_P)

    bf = jnp.bfloat16
    wpad = lambda w: jnp.pad(w.T, ((0, D_P - D), (0, 0))).astype(bf)
    logit = _rnn_call(
        x,
        wpad(W_ih_f), W_hh_f.T.astype(bf), b_ih_f.reshape(1, -1),
        b_hh_f.reshape(1, -1),
        wpad(W_ih_b), W_hh_b.T.astype(bf), b_ih_b.reshape(1, -1),
        b_hh_b.reshape(1, -1),
        W1.T, b1.reshape(1, -1), W2.T, b2.reshape(1, -1))
    return logit


# speedup vs baseline: 1.0868x; 1.0169x over previous
import functools

import jax
import jax.numpy as jnp
from jax import lax
from jax.experimental import pallas as pl
from jax.experimental.pallas import tpu as pltpu
from jax.experimental.pallas import tpu_sc as plsc

N_FEAT = 26
VOCAB = 100001
VOCAB_P = 100032
EMB = 16
B = 1024
L = 200
H = 128
D = N_FEAT * EMB
TOP = 32
D_P = 512

NC = 2
NS = 16
NW = NC * NS
N_TOT = L * B * N_FEAT
SLOT = 32
N_OUT = L * B * SLOT
ROWS_PER_W = N_TOT // NW
KK = 13
CSZ = KK * 128
N_CHUNK = ROWS_PER_W // CSZ
CHUNKS_PER_F = (B * L) // CSZ


def _sc_gather_body(table_hbm, idx_hbm, dst_hbm, out_hbm,
                    idx_v, dst_v, rows_v, sem, sem_s):
    wid = lax.axis_index("s") * NC + lax.axis_index("c")
    row0 = wid * (ROWS_PER_W // 128)

    H1 = KK // 2

    def chunk(cj, carry):
        f = (wid * N_CHUNK + cj) // CHUNKS_PER_F
        pltpu.sync_copy(idx_hbm.at[pl.ds(row0 + cj * KK, KK)], idx_v)
        pltpu.sync_copy(dst_hbm.at[pl.ds(row0 + cj * KK, KK)], dst_v)

        def gather(jj):
            return pltpu.async_copy(
                table_hbm.at[f].at[idx_v.at[jj]],
                rows_v.at[pl.ds(jj * 128, 128)], sem)

        def scatter(jj):
            return pltpu.async_copy(
                rows_v.at[pl.ds(jj * 128, 128)],
                out_hbm.at[dst_v.at[jj]], sem_s)

        g = [gather(jj) for jj in range(KK)]
        for h in g:
            h.wait()
        s = [scatter(jj) for jj in range(KK)]
        for h in s:
            h.wait()
        return carry

    lax.fori_loop(0, N_CHUNK, chunk, 0)


def _sc_gather(flat_table, idx_w, dst_w):
    mesh = plsc.VectorSubcoreMesh(core_axis_name="c", subcore_axis_name="s")
    k = functools.partial(
        pl.kernel,
        mesh=mesh,
        out_type=jax.ShapeDtypeStruct((N_OUT, EMB), jnp.float32),
        scratch_types=[
            pltpu.VMEM((KK, 128), jnp.int32),
            pltpu.VMEM((KK, 128), jnp.int32),
            pltpu.VMEM((CSZ, EMB), jnp.float32),
            pltpu.SemaphoreType.DMA,
            pltpu.SemaphoreType.DMA,
        ],
        compiler_params=pltpu.CompilerParams(use_tc_tiling_on_sc=False),
    )(_sc_gather_body)
    return k(flat_table, idx_w, dst_w)


def _rnn_body(xf_ref, xb_ref, wihf, whhf, bihf, bhhf,
              wihb, whhb, bihb, bhhb, w1t, b1, w2t, b2,
              out_ref, hf, hb, mxf, mxb, smf, smb):
    t = pl.program_id(0)

    @pl.when(t == 0)
    def _init():
        z = jnp.zeros((B, H), jnp.float32)
        ninf = jnp.full((B, H), -jnp.inf, jnp.float32)
        hf[...] = z
        hb[...] = z
        smf[...] = z
        smb[...] = z
        mxf[...] = ninf
        mxb[...] = ninf

    def gru_step(x, h, wih, whh, bih, bhh):
        gi = jnp.dot(x, wih[...], preferred_element_type=jnp.float32) + bih[...]
        gh = jnp.dot(h, whh[...], preferred_element_type=jnp.float32) + bhh[...]
        i_r, i_z, i_n = gi[:, :H], gi[:, H:2 * H], gi[:, 2 * H:]
        h_r, h_z, h_n = gh[:, :H], gh[:, H:2 * H], gh[:, 2 * H:]
        r = jax.nn.sigmoid(i_r + h_r)
        z = jax.nn.sigmoid(i_z + h_z)
        n = jnp.tanh(i_n + r * h_n)
        return (1.0 - z) * n + z * h

    lane = lax.broadcasted_iota(jnp.int32, (B, D_P), 1)
    xf = jnp.where(lane < D, xf_ref[0], 0.0)
    xb = jnp.where(lane < D, xb_ref[0], 0.0)
    hf_new = gru_step(xf, hf[...], wihf, whhf, bihf, bhhf)
    hb_new = gru_step(xb, hb[...], wihb, whhb, bihb, bhhb)
    hf[...] = hf_new
    hb[...] = hb_new
    mxf[...] = jnp.maximum(mxf[...], hf_new)
    mxb[...] = jnp.maximum(mxb[...], hb_new)
    smf[...] = smf[...] + hf_new
    smb[...] = smb[...] + hb_new

    @pl.when(t == L - 1)
    def _final():
        inv_l = jnp.float32(1.0 / L)
        combined = jnp.concatenate(
            [mxf[...], mxb[...], smf[...] * inv_l, smb[...] * inv_l], axis=1)
        h1 = jnp.maximum(
            jnp.dot(combined, w1t[...], preferred_element_type=jnp.float32)
            + b1[...], 0.0)
        logit = jnp.dot(h1, w2t[...], preferred_element_type=jnp.float32) + b2[...]
        out_ref[...] = logit


def _rnn_call(x, wihf_t, whhf_t, bihf, bhhf, wihb_t, whhb_t, bihb, bhhb,
              w1t, b1, w2t, b2, interpret=False):
    full = lambda s: pl.BlockSpec(s, lambda t: (0,) * len(s))
    return pl.pallas_call(
        _rnn_body,
        grid=(L,),
        in_specs=[
            pl.BlockSpec((1, B, D_P), lambda t: (t, 0, 0)),
            pl.BlockSpec((1, B, D_P), lambda t: (L - 1 - t, 0, 0)),
            full((D_P, 3 * H)), full((H, 3 * H)), full((1, 3 * H)), full((1, 3 * H)),
            full((D_P, 3 * H)), full((H, 3 * H)), full((1, 3 * H)), full((1, 3 * H)),
            full((4 * H, TOP)), full((1, TOP)), full((TOP, 1)), full((1, 1)),
        ],
        out_specs=pl.BlockSpec((B, 1), lambda t: (0, 0)),
        out_shape=jax.ShapeDtypeStruct((B, 1), jnp.float32),
        scratch_shapes=[pltpu.VMEM((B, H), jnp.float32)] * 6,
        compiler_params=pltpu.CompilerParams(
            dimension_semantics=("arbitrary",)),
        interpret=interpret,
    )(x, x, wihf_t, whhf_t, bihf, bhhf, wihb_t, whhb_t, bihb, bhhb,
      w1t, b1, w2t, b2)


def kernel(transactions_cat_features, emb_tables, W_ih_f, W_hh_f, b_ih_f,
           b_hh_f, W_ih_b, W_hh_b, b_ih_b, b_hh_b, W1, b1, W2, b2):
    table_p = jnp.pad(emb_tables, ((0, 0), (0, VOCAB_P - VOCAB), (0, 0)))
    f_ax = jnp.arange(N_FEAT, dtype=jnp.int32)[:, None, None]
    b_ax = jnp.arange(B, dtype=jnp.int32)[None, :, None]
    t_ax = jnp.arange(L, dtype=jnp.int32)[None, None, :]
    dst_nat = t_ax * (B * SLOT) + b_ax * SLOT + f_ax
    idx_w = transactions_cat_features.reshape(N_TOT // 128, 128)
    dst_w = jnp.broadcast_to(
        dst_nat, transactions_cat_features.shape).reshape(N_TOT // 128, 128)

    x = _sc_gather(table_p, idx_w, dst_w).reshape(L, B, D_P)

    wpad = lambda w: jnp.pad(w.T, ((0, D_P - D), (0, 0)))
    logit = _rnn_call(
        x,
        wpad(W_ih_f), W_hh_f.T, b_ih_f.reshape(1, -1), b_hh_f.reshape(1, -1),
        wpad(W_ih_b), W_hh_b.T, b_ih_b.reshape(1, -1), b_hh_b.reshape(1, -1),
        W1.T, b1.reshape(1, -1), W2.T, b2.reshape(1, -1))
    return logit
